# baseline (speedup 1.0000x reference)
import functools

import jax
import jax.numpy as jnp
from jax import lax
from jax.experimental import pallas as pl
from jax.experimental.pallas import tpu as pltpu
from jax.experimental.pallas import tpu_sc as plsc

N_CORES = 2
N_SUBCORES = 16
LANES = 16
N_WORKERS = N_CORES * N_SUBCORES
BATCH = 16384
FACTORS = 16
BPW = BATCH // N_WORKERS
GROUP = 16
GROUPS = BPW // GROUP
PAIRS = GROUPS // 2


def _mf_body(user_table_t, movie_table_t, bias_user, bias_movie, bias,
             user, movie, out,
             uidx_v, midx_v, blk0_v, blk1_v, urows_v,
             bu_v, bm_v, bias_v, out_v, sem0, sem1, bsem):
    wid = lax.axis_index("s") * N_CORES + lax.axis_index("c")
    base = wid * BPW

    pltpu.sync_copy(user.at[pl.ds(base, BPW)], uidx_v)
    pltpu.sync_copy(movie.at[pl.ds(base, BPW)], midx_v)
    pltpu.sync_copy(bias, bias_v)

    bcopies = [
        pltpu.async_copy(bias_user.at[uidx_v], bu_v, bsem),
        pltpu.async_copy(bias_movie.at[midx_v], bm_v, bsem),
    ]

    bias_vec = bias_v[...]
    iota = lax.iota(jnp.int32, LANES)

    def fire(table, idx_v, g, blk, sem):
        ridx = idx_v[pl.ds(g * GROUP, LANES)]
        tiles = (ridx >> 7) * 128
        for j in range(GROUP):
            off = pl.multiple_of(tiles[j], 128)
            pltpu.async_copy(table.at[:, pl.ds(off, 128)], blk.at[j], sem)

    def drain(table, blk, sem):
        for j in range(GROUP):
            pltpu.make_async_copy(
                table.at[:, pl.ds(0, 128)], blk.at[j], sem).wait()

    def extract_u(g, blk):
        sl = pl.ds(g * GROUP, LANES)
        lanes = uidx_v[sl] & 127
        for f in range(FACTORS):
            fvec = (iota & 0) + f
            urows_v[f, sl] = plsc.load_gather(blk, [iota, fvec, lanes])

    def extract_m(g, blk):
        sl = pl.ds(g * GROUP, LANES)
        lanes = midx_v[sl] & 127
        acc = bu_v[sl] + bm_v[sl] + bias_vec
        for f in range(FACTORS):
            fvec = (iota & 0) + f
            mm = plsc.load_gather(blk, [iota, fvec, lanes])
            acc = acc + urows_v[f, sl] * mm
        out_v[sl] = acc

    def make_phase(table, idx_v, extract):
        def body(p, carry):
            g0 = 2 * p
            g1 = g0 + 1
            fire(table, idx_v, g1, blk1_v, sem1)
            drain(table, blk0_v, sem0)
            extract(g0, blk0_v)

            @pl.when(g0 + 2 < GROUPS)
            def _():
                fire(table, idx_v, g0 + 2, blk0_v, sem0)

            drain(table, blk1_v, sem1)
            extract(g1, blk1_v)
            return carry

        fire(table, idx_v, 0, blk0_v, sem0)
        lax.fori_loop(0, PAIRS, body, 0)

    make_phase(user_table_t, uidx_v, extract_u)
    for c in bcopies:
        c.wait()
    make_phase(movie_table_t, midx_v, extract_m)

    pltpu.sync_copy(out_v, out.at[pl.ds(base, BPW)])


@jax.jit
def _mf(user_table_t, movie_table_t, bias_user, bias_movie, bias,
        user, movie):
    run = functools.partial(
        pl.kernel,
        mesh=plsc.VectorSubcoreMesh(core_axis_name="c", subcore_axis_name="s"),
        out_type=jax.ShapeDtypeStruct((BATCH,), jnp.float32),
        scratch_types=[
            pltpu.VMEM((BPW,), jnp.int32),
            pltpu.VMEM((BPW,), jnp.int32),
            pltpu.VMEM((GROUP, FACTORS, 128), jnp.float32),
            pltpu.VMEM((GROUP, FACTORS, 128), jnp.float32),
            pltpu.VMEM((FACTORS, BPW), jnp.float32),
            pltpu.VMEM((BPW,), jnp.float32),
            pltpu.VMEM((BPW,), jnp.float32),
            pltpu.VMEM((LANES,), jnp.float32),
            pltpu.VMEM((BPW,), jnp.float32),
            pltpu.SemaphoreType.DMA,
            pltpu.SemaphoreType.DMA,
            pltpu.SemaphoreType.DMA,
        ],
        compiler_params=pltpu.CompilerParams(needs_layout_passes=False),
    )(_mf_body)
    return run(user_table_t, movie_table_t, bias_user, bias_movie, bias,
               user, movie)


def kernel(user_table, movie_table, bias_user, bias_movie, bias, user, movie):
    return _mf(
        user_table.T,
        movie_table.T,
        bias_user.reshape(-1),
        bias_movie.reshape(-1),
        jnp.broadcast_to(bias, (LANES,)),
        user.astype(jnp.int32),
        movie.astype(jnp.int32),
    )


# speedup vs baseline: 1.0139x; 1.0139x over previous
import functools

import jax
import jax.numpy as jnp
from jax import lax
from jax.experimental import pallas as pl
from jax.experimental.pallas import tpu as pltpu
from jax.experimental.pallas import tpu_sc as plsc

N_CORES = 2
N_SUBCORES = 16
LANES = 16
N_WORKERS = N_CORES * N_SUBCORES
BATCH = 16384
FACTORS = 16
BPW = BATCH // N_WORKERS
GROUP = 16
GROUPS = BPW // GROUP


def _mf_body(user_table_t, movie_table_t, bias_user, bias_movie, bias,
             user, movie, out,
             uidx_v, midx_v, ublk_v, mblk_v,
             bu_v, bm_v, bias_v, out_v, sem, bsem):
    wid = lax.axis_index("s") * N_CORES + lax.axis_index("c")
    base = wid * BPW

    pltpu.sync_copy(user.at[pl.ds(base, BPW)], uidx_v)
    pltpu.sync_copy(movie.at[pl.ds(base, BPW)], midx_v)
    pltpu.sync_copy(bias, bias_v)

    bcopies = [
        pltpu.async_copy(bias_user.at[uidx_v], bu_v, bsem),
        pltpu.async_copy(bias_movie.at[midx_v], bm_v, bsem),
    ]

    bias_vec = bias_v[...]
    iota = lax.iota(jnp.int32, LANES)

    def group(g, carry):
        sl = pl.ds(g * GROUP, LANES)
        ridx_u = uidx_v[sl]
        ridx_m = midx_v[sl]
        tile_u = (ridx_u >> 7) * 128
        tile_m = (ridx_m >> 7) * 128
        copies = []
        for j in range(GROUP):
            ou = pl.multiple_of(tile_u[j], 128)
            om = pl.multiple_of(tile_m[j], 128)
            copies.append(pltpu.async_copy(
                user_table_t.at[:, pl.ds(ou, 128)], ublk_v.at[j], sem))
            copies.append(pltpu.async_copy(
                movie_table_t.at[:, pl.ds(om, 128)], mblk_v.at[j], sem))
        for c in copies:
            c.wait()

        lanes_u = ridx_u & 127
        lanes_m = ridx_m & 127
        acc = bias_vec
        for f in range(FACTORS):
            fvec = (iota & 0) + f
            uu = plsc.load_gather(ublk_v, [iota, fvec, lanes_u])
            mm = plsc.load_gather(mblk_v, [iota, fvec, lanes_m])
            acc = acc + uu * mm
        out_v[sl] = acc
        return carry

    lax.fori_loop(0, GROUPS, group, 0)

    for c in bcopies:
        c.wait()

    def addbias(g, carry):
        sl = pl.ds(g * GROUP, LANES)
        out_v[sl] = out_v[sl] + bu_v[sl] + bm_v[sl]
        return carry

    lax.fori_loop(0, GROUPS, addbias, 0)

    pltpu.sync_copy(out_v, out.at[pl.ds(base, BPW)])


@jax.jit
def _mf(user_table_t, movie_table_t, bias_user, bias_movie, bias,
        user, movie):
    run = functools.partial(
        pl.kernel,
        mesh=plsc.VectorSubcoreMesh(core_axis_name="c", subcore_axis_name="s"),
        out_type=jax.ShapeDtypeStruct((BATCH,), jnp.float32),
        scratch_types=[
            pltpu.VMEM((BPW,), jnp.int32),
            pltpu.VMEM((BPW,), jnp.int32),
            pltpu.VMEM((GROUP, FACTORS, 128), jnp.float32),
            pltpu.VMEM((GROUP, FACTORS, 128), jnp.float32),
            pltpu.VMEM((BPW,), jnp.float32),
            pltpu.VMEM((BPW,), jnp.float32),
            pltpu.VMEM((LANES,), jnp.float32),
            pltpu.VMEM((BPW,), jnp.float32),
            pltpu.SemaphoreType.DMA,
            pltpu.SemaphoreType.DMA,
        ],
        compiler_params=pltpu.CompilerParams(needs_layout_passes=False),
    )(_mf_body)
    return run(user_table_t, movie_table_t, bias_user, bias_movie, bias,
               user, movie)


def kernel(user_table, movie_table, bias_user, bias_movie, bias, user, movie):
    return _mf(
        user_table.T,
        movie_table.T,
        bias_user.reshape(-1),
        bias_movie.reshape(-1),
        jnp.broadcast_to(bias, (LANES,)),
        user.astype(jnp.int32),
        movie.astype(jnp.int32),
    )
